# baseline (speedup 1.0000x reference)
import jax
import jax.numpy as jnp
from jax import lax
from jax.experimental import pallas as pl
from jax.experimental.pallas import tpu as pltpu
from jax.experimental.pallas import tpu_sc as plsc

B = 16384
TD = 32
NNUM = 8
VD = 128
QED = 32

NUM_TABLES = 5
NW = 32
BPW = B // NW


def _gather1_kernel(t4, idx, idx4, out, idx_v, idx4_v, gbuf, ebuf, gsem):
  nc = 2
  wid = lax.axis_index("s") * nc + lax.axis_index("c")
  base = wid * BPW

  pltpu.sync_copy(idx.at[pl.ds(base, BPW)], idx_v)
  pltpu.sync_copy(idx4.at[pl.ds(base, BPW)], idx4_v)

  HALF = BPW // 2
  for c in range(2):
    pltpu.async_copy(
        t4.at[idx4_v.at[pl.ds(c * HALF, HALF)]], gbuf, gsem).wait()

    @pl.loop(0, HALF, step=16)
    def _extract(i):
      v = idx_v[pl.ds(c * HALF + i, 16)]
      for j in range(16):
        off = (v[j] & 3) * TD
        row = gbuf.at[i + j]
        erow = ebuf.at[c * HALF + i + j]
        erow[pl.ds(0, 16)] = row[pl.ds(off, 16)]
        erow[pl.ds(16, 16)] = row[pl.ds(off + 16, 16)]

  pltpu.sync_copy(ebuf, out.at[pl.ds(base, BPW)])


def _sc_gather_one(t4, idx, idx4):
  mesh = plsc.VectorSubcoreMesh(core_axis_name="c", subcore_axis_name="s")
  fn = pl.kernel(
      _gather1_kernel,
      out_type=jax.ShapeDtypeStruct((B, TD), jnp.float32),
      mesh=mesh,
      scratch_types=[
          pltpu.VMEM((BPW,), jnp.int32),
          pltpu.VMEM((BPW,), jnp.int32),
          pltpu.VMEM((BPW // 2, 128), jnp.float32),
          pltpu.VMEM((BPW, TD), jnp.float32),
          pltpu.SemaphoreType.DMA,
      ],
  )
  return fn(t4, idx, idx4)


def _mlp_kernel(ea, eb, ec, ed, eq, num, vec,
                nw1, nb1, nw2, nb2, vw, vb, mw1, mb1, mw2, mb2,
                out):
  h = jnp.maximum(
      jnp.dot(num[...], nw1[...], preferred_element_type=jnp.float32)
      + nb1[...], 0.0)
  h = jnp.dot(h, nw2[...], preferred_element_type=jnp.float32) + nb2[...]
  v = jnp.dot(vec[...], vw[...], preferred_element_type=jnp.float32) + vb[...]
  feat = jnp.concatenate(
      [ea[...], eb[...], ec[...], ed[...], eq[...], h, v], axis=1)
  g = jnp.maximum(
      jnp.dot(feat, mw1[...], preferred_element_type=jnp.float32) + mb1[...],
      0.0)
  out[...] = (
      jnp.dot(g, mw2[...], preferred_element_type=jnp.float32) + mb2[...])


def _tc_mlp(emb_a, emb_b, emb_c, emb_d, emb_q, numericals, vec_emb,
            num_W1, num_b1, num_W2, num_b2, vec_W, vec_b,
            merge_W1, merge_b1, merge_W2, merge_b2):
  BB = 2048
  grid = (B // BB,)

  def batch_spec(width):
    return pl.BlockSpec((BB, width), lambda i: (i, 0))

  def full_spec(shape):
    return pl.BlockSpec(shape, lambda i: tuple(0 for _ in shape))

  return pl.pallas_call(
      _mlp_kernel,
      grid=grid,
      in_specs=[
          batch_spec(TD), batch_spec(TD), batch_spec(TD), batch_spec(TD),
          batch_spec(TD), batch_spec(NNUM), batch_spec(VD),
          full_spec(num_W1.shape), full_spec(num_b1.shape),
          full_spec(num_W2.shape), full_spec(num_b2.shape),
          full_spec(vec_W.shape), full_spec(vec_b.shape),
          full_spec(merge_W1.shape), full_spec(merge_b1.shape),
          full_spec(merge_W2.shape), full_spec(merge_b2.shape),
      ],
      out_specs=batch_spec(QED),
      out_shape=jax.ShapeDtypeStruct((B, QED), jnp.float32),
  )(emb_a, emb_b, emb_c, emb_d, emb_q, numericals, vec_emb,
    num_W1, num_b1, num_W2, num_b2, vec_W, vec_b,
    merge_W1, merge_b1, merge_W2, merge_b2)


def kernel(query_id, cat_a, cat_b, cat_c, cat_d, numericals, vec_emb,
           query_table, ct_a, ct_b, ct_c, ct_d,
           num_W1, num_b1, num_W2, num_b2,
           vec_W, vec_b,
           merge_W1, merge_b1, merge_W2, merge_b2):
  ids = [x.astype(jnp.int32)
         for x in (cat_a, cat_b, cat_c, cat_d, query_id)]
  ids4 = [x >> 2 for x in ids]
  packed = [t.reshape(t.shape[0] // 4, 128)
            for t in (ct_a, ct_b, ct_c, ct_d, query_table)]

  embs = [_sc_gather_one(packed[f], ids[f], ids4[f])
          for f in range(NUM_TABLES)]
  ea, eb, ec, ed, eq = embs

  return _tc_mlp(
      ea, eb, ec, ed, eq, numericals, vec_emb,
      num_W1, num_b1.reshape(1, -1), num_W2, num_b2.reshape(1, -1),
      vec_W, vec_b.reshape(1, -1),
      merge_W1, merge_b1.reshape(1, -1), merge_W2, merge_b2.reshape(1, -1))


# speedup vs baseline: 1.1082x; 1.1082x over previous
import jax
import jax.numpy as jnp
from jax import lax
from jax.experimental import pallas as pl
from jax.experimental.pallas import tpu as pltpu
from jax.experimental.pallas import tpu_sc as plsc

B = 16384
TD = 32
NNUM = 8
VD = 128
QED = 32

NW = 32
BPW = B // NW

_MESH = dict(core_axis_name="c", subcore_axis_name="s")


def _worker_base():
  wid = lax.axis_index("s") * 2 + lax.axis_index("c")
  return wid * BPW


def _qgather_kernel(t4, idx, idx4, out, idx_v, idx4_v, gbuf, ebuf, gsem):
  base = _worker_base()
  pltpu.sync_copy(idx.at[pl.ds(base, BPW)], idx_v)
  pltpu.sync_copy(idx4.at[pl.ds(base, BPW)], idx4_v)

  HALF = BPW // 2
  for c in range(2):
    pltpu.async_copy(
        t4.at[idx4_v.at[pl.ds(c * HALF, HALF)]], gbuf, gsem).wait()

    @pl.loop(0, HALF, step=16)
    def _extract(i):
      v = idx_v[pl.ds(c * HALF + i, 16)]
      for j in range(16):
        off = (v[j] & 3) * TD
        row = gbuf.at[i + j]
        erow = ebuf.at[c * HALF + i + j]
        erow[pl.ds(0, 16)] = row[pl.ds(off, 16)]
        erow[pl.ds(16, 16)] = row[pl.ds(off + 16, 16)]

  pltpu.sync_copy(ebuf, out.at[pl.ds(base, BPW)])


def _sc_gather_query(t4, idx, idx4):
  fn = pl.kernel(
      _qgather_kernel,
      out_type=jax.ShapeDtypeStruct((B, TD), jnp.float32),
      mesh=plsc.VectorSubcoreMesh(**_MESH),
      scratch_types=[
          pltpu.VMEM((BPW,), jnp.int32),
          pltpu.VMEM((BPW,), jnp.int32),
          pltpu.VMEM((BPW // 2, 128), jnp.float32),
          pltpu.VMEM((BPW, TD), jnp.float32),
          pltpu.SemaphoreType.DMA,
      ],
  )
  return fn(t4, idx, idx4)


def _cgather_kernel(ta, tb, tc, td,
                    ia, ib, ic, id_,
                    oa, ob, oc, od,
                    idx_v, rows_v, gsem):
  base = _worker_base()
  tables = (ta, tb, tc, td)
  idxs = (ia, ib, ic, id_)
  outs = (oa, ob, oc, od)

  for f in range(4):
    pltpu.sync_copy(idxs[f].at[pl.ds(base, BPW)], idx_v)
    table = tables[f]

    @pl.loop(0, BPW, step=16)
    def _rows(i):
      v = idx_v[pl.ds(i, 16)]
      for j in range(16):
        pltpu.async_copy(
            table.at[pl.ds(v[j], 1)], rows_v.at[pl.ds(i + j, 1)], gsem)

    pltpu.make_async_copy(table.at[pl.ds(0, BPW)], rows_v, gsem).wait()
    pltpu.sync_copy(rows_v, outs[f].at[pl.ds(base, BPW)])


def _sc_gather_cats(tables, idxs):
  fn = pl.kernel(
      _cgather_kernel,
      out_type=tuple(
          jax.ShapeDtypeStruct((B, TD), jnp.float32) for _ in range(4)),
      mesh=plsc.VectorSubcoreMesh(**_MESH),
      scratch_types=[
          pltpu.VMEM((BPW,), jnp.int32),
          pltpu.VMEM((BPW, TD), jnp.float32),
          pltpu.SemaphoreType.DMA,
      ],
  )
  return fn(*tables, *idxs)


def _mlp_kernel(ea, eb, ec, ed, eq, num, vec,
                nw1, nb1, nw2, nb2, vw, vb, mw1, mb1, mw2, mb2,
                out):
  h = jnp.maximum(
      jnp.dot(num[...], nw1[...], preferred_element_type=jnp.float32)
      + nb1[...], 0.0)
  h = jnp.dot(h, nw2[...], preferred_element_type=jnp.float32) + nb2[...]
  v = jnp.dot(vec[...], vw[...], preferred_element_type=jnp.float32) + vb[...]
  feat = jnp.concatenate(
      [ea[...], eb[...], ec[...], ed[...], eq[...], h, v], axis=1)
  g = jnp.maximum(
      jnp.dot(feat, mw1[...], preferred_element_type=jnp.float32) + mb1[...],
      0.0)
  out[...] = (
      jnp.dot(g, mw2[...], preferred_element_type=jnp.float32) + mb2[...])


def _tc_mlp(emb_a, emb_b, emb_c, emb_d, emb_q, numericals, vec_emb,
            num_W1, num_b1, num_W2, num_b2, vec_W, vec_b,
            merge_W1, merge_b1, merge_W2, merge_b2):
  BB = 2048
  grid = (B // BB,)

  def batch_spec(width):
    return pl.BlockSpec((BB, width), lambda i: (i, 0))

  def full_spec(shape):
    return pl.BlockSpec(shape, lambda i: tuple(0 for _ in shape))

  return pl.pallas_call(
      _mlp_kernel,
      grid=grid,
      in_specs=[
          batch_spec(TD), batch_spec(TD), batch_spec(TD), batch_spec(TD),
          batch_spec(TD), batch_spec(NNUM), batch_spec(VD),
          full_spec(num_W1.shape), full_spec(num_b1.shape),
          full_spec(num_W2.shape), full_spec(num_b2.shape),
          full_spec(vec_W.shape), full_spec(vec_b.shape),
          full_spec(merge_W1.shape), full_spec(merge_b1.shape),
          full_spec(merge_W2.shape), full_spec(merge_b2.shape),
      ],
      out_specs=batch_spec(QED),
      out_shape=jax.ShapeDtypeStruct((B, QED), jnp.float32),
  )(emb_a, emb_b, emb_c, emb_d, emb_q, numericals, vec_emb,
    num_W1, num_b1, num_W2, num_b2, vec_W, vec_b,
    merge_W1, merge_b1, merge_W2, merge_b2)


def kernel(query_id, cat_a, cat_b, cat_c, cat_d, numericals, vec_emb,
           query_table, ct_a, ct_b, ct_c, ct_d,
           num_W1, num_b1, num_W2, num_b2,
           vec_W, vec_b,
           merge_W1, merge_b1, merge_W2, merge_b2):
  qid = query_id.astype(jnp.int32)
  cids = [x.astype(jnp.int32) for x in (cat_a, cat_b, cat_c, cat_d)]

  q4 = query_table.reshape(query_table.shape[0] // 4, 128)
  eq = _sc_gather_query(q4, qid, qid >> 2)
  ea, eb, ec, ed = _sc_gather_cats((ct_a, ct_b, ct_c, ct_d), cids)

  return _tc_mlp(
      ea, eb, ec, ed, eq, numericals, vec_emb,
      num_W1, num_b1.reshape(1, -1), num_W2, num_b2.reshape(1, -1),
      vec_W, vec_b.reshape(1, -1),
      merge_W1, merge_b1.reshape(1, -1), merge_W2, merge_b2.reshape(1, -1))
